# baseline (speedup 1.0000x reference)
import functools

import jax
import jax.numpy as jnp
from jax import lax
from jax.experimental import pallas as pl
from jax.experimental.pallas import tpu as pltpu
from jax.experimental.pallas import tpu_sc as plsc

_NSEG = 196
_SEGP = 256
_NQ = 4
_LANES = 16
_NW = 32
_EMB = 768


def _tc_fuse_weights(W_proj, b_proj2, W_gcn, b_gcn2):
    def tc_kernel(wp_ref, bp_ref, wg_ref, bg_ref, o_ref):
        wg = wg_ref[...]
        o_ref[0:3, :] = jnp.dot(wp_ref[...], wg,
                                preferred_element_type=jnp.float32)
        o_ref[3:4, :] = jnp.dot(bp_ref[...], wg,
                                preferred_element_type=jnp.float32) + bg_ref[...]

    return pl.pallas_call(
        tc_kernel,
        out_shape=jax.ShapeDtypeStruct((4, _EMB), jnp.float32),
    )(W_proj, b_proj2, W_gcn, b_gcn2)


def _sc_tokenize(img_flat, seg_flat, wf_flat, B, C, HW):
    ppw = (B * HW) // _NW
    wpb = _NW // B
    groups = ppw // _LANES
    rowstr = 264
    live = 208
    qstride = _LANES * rowstr
    accsz = _NQ * qstride
    psz = _NQ * _SEGP
    rows_pw = 25

    mesh = plsc.VectorSubcoreMesh(
        core_axis_name="c", subcore_axis_name="s", num_cores=2, num_subcores=16)

    @functools.partial(
        pl.kernel,
        mesh=mesh,
        compiler_params=pltpu.CompilerParams(needs_layout_passes=False),
        out_type=jax.ShapeDtypeStruct((B * _NSEG * _EMB,), jnp.float32),
        scratch_types=[
            pltpu.VMEM((ppw,), jnp.int32),
            pltpu.VMEM((ppw,), jnp.float32),
            pltpu.VMEM((ppw,), jnp.float32),
            pltpu.VMEM((ppw,), jnp.float32),
            pltpu.VMEM((accsz,), jnp.float32),
            pltpu.VMEM((psz,), jnp.float32),
            pltpu.VMEM((_NQ * _EMB,), jnp.float32),
            pltpu.VMEM((wpb, psz), jnp.float32),
            pltpu.VMEM((rows_pw * _EMB,), jnp.float32),
            pltpu.VMEM_SHARED((_LANES, psz), jnp.float32),
            pltpu.SemaphoreType.DMA,
        ],
    )
    def sc_kernel(img_hbm, seg_hbm, wf_hbm, out_hbm,
                  seg_v, ch0, ch1, ch2, acc, outv, wfv, comb, rep, shared, sem):
        cid = lax.axis_index("c")
        sid = lax.axis_index("s")
        b = cid * 2 + sid // wpb
        j = sid % wpb
        pix0 = j * ppw
        cps = [
            pltpu.async_copy(seg_hbm.at[pl.ds(b * HW + pix0, ppw)], seg_v, sem),
            pltpu.async_copy(img_hbm.at[pl.ds((b * C + 0) * HW + pix0, ppw)], ch0, sem),
            pltpu.async_copy(img_hbm.at[pl.ds((b * C + 1) * HW + pix0, ppw)], ch1, sem),
            pltpu.async_copy(img_hbm.at[pl.ds((b * C + 2) * HW + pix0, ppw)], ch2, sem),
            pltpu.async_copy(wf_hbm.at[pl.ds(0, _NQ * _EMB)], wfv, sem),
        ]

        zeros = jnp.zeros((_LANES,), jnp.float32)

        def zbody(i, carry):
            q = i // _LANES
            l = i % _LANES
            base = q * qstride + l * rowstr
            for t in range(live // _LANES):
                acc[pl.ds(base + t * _LANES, _LANES)] = zeros
            return carry

        lax.fori_loop(0, _NQ * _LANES, zbody, 0)

        for cp in cps:
            cp.wait()

        lanebase = lax.iota(jnp.int32, _LANES) * rowstr
        ones = jnp.ones((_LANES,), jnp.float32)
        GUN = 8

        def body(g, carry):
            base = g * (GUN * _LANES)
            idxs = []
            for u in range(GUN):
                o = base + u * _LANES
                idxs.append(lanebase + seg_v[pl.ds(o, _LANES)])
            for q, ch in enumerate((ch0, ch1, ch2, None)):
                qb = q * qstride
                for u in range(GUN):
                    o = base + u * _LANES
                    val = ones if ch is None else ch[pl.ds(o, _LANES)]
                    plsc.addupdate_scatter(acc, [idxs[u] + qb], val)
            return carry

        lax.fori_loop(0, groups // GUN, body, 0)

        for q in range(_NQ):
            qb = q * qstride
            for t in range(live // _LANES):
                vs = [acc[pl.ds(qb + l * rowstr + t * _LANES, _LANES)]
                      for l in range(_LANES)]
                while len(vs) > 1:
                    vs = [vs[i] + vs[i + 1] for i in range(0, len(vs), 2)]
                outv[pl.ds(q * _SEGP + t * _LANES, _LANES)] = vs[0]

        pltpu.sync_copy(outv, shared.at[sid])
        plsc.subcore_barrier()
        g0 = (sid // wpb) * wpb
        pltpu.sync_copy(shared.at[pl.ds(g0, wpb)], comb)

        def tot(q, t):
            vs = [comb[r, pl.ds(q * _SEGP + t * _LANES, _LANES)]
                  for r in range(wpb)]
            while len(vs) > 1:
                vs = [vs[i] + vs[i + 1] for i in range(0, len(vs), 2)]
            return vs[0]

        def tree(vs):
            while len(vs) > 1:
                nxt = [vs[i] + vs[i + 1] for i in range(0, len(vs) - 1, 2)]
                if len(vs) % 2:
                    nxt.append(vs[-1])
                vs = nxt
            return vs[0]

        cnts = [jnp.maximum(tot(3, t), 1.0) for t in range(live // _LANES)]
        a = []
        for c in range(3):
            ms = tree([tot(c, t) / cnts[t] for t in range(live // _LANES)])
            a.append(lax.reduce_sum(ms, (0,)) * (1.0 / float(_NSEG)))

        us = []
        for t in range(_EMB // _LANES):
            us.append(a[0] * wfv[pl.ds(0 * _EMB + t * _LANES, _LANES)]
                      + a[1] * wfv[pl.ds(1 * _EMB + t * _LANES, _LANES)]
                      + a[2] * wfv[pl.ds(2 * _EMB + t * _LANES, _LANES)]
                      + wfv[pl.ds(3 * _EMB + t * _LANES, _LANES)])

        def rbody(r, carry):
            base = r * _EMB
            for t in range(_EMB // _LANES):
                rep[pl.ds(base + t * _LANES, _LANES)] = us[t]
            return carry

        lax.fori_loop(0, rows_pw, rbody, 0)

        base_row = jnp.minimum(j * rows_pw, _NSEG - rows_pw)
        pltpu.sync_copy(
            rep, out_hbm.at[pl.ds((b * _NSEG + base_row) * _EMB,
                                  rows_pw * _EMB)])

    return sc_kernel(img_flat, seg_flat, wf_flat)


def kernel(img, segments, W_proj, b_proj, W_gcn, b_gcn):
    B, C, H, W = img.shape
    HW = H * W
    img_flat = img.reshape(B * C * HW)
    seg_flat = segments.reshape(B * HW).astype(jnp.int32)
    wf = _tc_fuse_weights(W_proj, b_proj.reshape(1, _EMB),
                          W_gcn, b_gcn.reshape(1, _EMB))
    out = _sc_tokenize(img_flat, seg_flat, wf.reshape(_NQ * _EMB), B, C, HW)
    return out.reshape(B, _NSEG, _EMB)


# speedup vs baseline: 1.1325x; 1.1325x over previous
import functools

import jax
import jax.numpy as jnp
from jax import lax
from jax.experimental import pallas as pl
from jax.experimental.pallas import tpu as pltpu
from jax.experimental.pallas import tpu_sc as plsc

_NSEG = 196
_SEGP = 256
_NQ = 4
_LANES = 16
_NW = 32
_EMB = 768


def _sc_segment_partials(img_flat, seg_flat, B, C, HW):
    ppw = (B * HW) // _NW
    wpb = _NW // B
    groups = ppw // _LANES
    rowstr = 264
    live = 208
    qstride = _LANES * rowstr
    accsz = _NQ * qstride
    outsz = _NQ * _SEGP

    mesh = plsc.VectorSubcoreMesh(
        core_axis_name="c", subcore_axis_name="s", num_cores=2, num_subcores=16)

    @functools.partial(
        pl.kernel,
        mesh=mesh,
        compiler_params=pltpu.CompilerParams(needs_layout_passes=False),
        out_type=jax.ShapeDtypeStruct((_NW * outsz,), jnp.float32),
        scratch_types=[
            pltpu.VMEM((ppw,), jnp.int32),
            pltpu.VMEM((ppw,), jnp.float32),
            pltpu.VMEM((ppw,), jnp.float32),
            pltpu.VMEM((ppw,), jnp.float32),
            pltpu.VMEM((accsz,), jnp.float32),
            pltpu.VMEM((outsz,), jnp.float32),
            pltpu.SemaphoreType.DMA,
            pltpu.SemaphoreType.DMA,
        ],
    )
    def sc_kernel(img_hbm, seg_hbm, out_hbm, seg_v, ch0, ch1, ch2, acc, outv,
                  sem_a, sem_b):
        cid = lax.axis_index("c")
        sid = lax.axis_index("s")
        wid = sid * 2 + cid
        b = wid // wpb
        j = wid % wpb
        pix0 = j * ppw
        hp = ppw // 2
        cps_a = [
            pltpu.async_copy(seg_hbm.at[pl.ds(b * HW + pix0, hp)],
                             seg_v.at[pl.ds(0, hp)], sem_a),
            pltpu.async_copy(img_hbm.at[pl.ds((b * C + 0) * HW + pix0, hp)],
                             ch0.at[pl.ds(0, hp)], sem_a),
            pltpu.async_copy(img_hbm.at[pl.ds((b * C + 1) * HW + pix0, hp)],
                             ch1.at[pl.ds(0, hp)], sem_a),
            pltpu.async_copy(img_hbm.at[pl.ds((b * C + 2) * HW + pix0, hp)],
                             ch2.at[pl.ds(0, hp)], sem_a),
        ]
        cps_b = [
            pltpu.async_copy(seg_hbm.at[pl.ds(b * HW + pix0 + hp, hp)],
                             seg_v.at[pl.ds(hp, hp)], sem_b),
            pltpu.async_copy(img_hbm.at[pl.ds((b * C + 0) * HW + pix0 + hp, hp)],
                             ch0.at[pl.ds(hp, hp)], sem_b),
            pltpu.async_copy(img_hbm.at[pl.ds((b * C + 1) * HW + pix0 + hp, hp)],
                             ch1.at[pl.ds(hp, hp)], sem_b),
            pltpu.async_copy(img_hbm.at[pl.ds((b * C + 2) * HW + pix0 + hp, hp)],
                             ch2.at[pl.ds(hp, hp)], sem_b),
        ]

        zeros = jnp.zeros((_LANES,), jnp.float32)

        def zbody(i, carry):
            q = i // _LANES
            l = i % _LANES
            base = q * qstride + l * rowstr
            for t in range(live // _LANES):
                acc[pl.ds(base + t * _LANES, _LANES)] = zeros
            return carry

        lax.fori_loop(0, _NQ * _LANES, zbody, 0)

        lanebase = lax.iota(jnp.int32, _LANES) * rowstr
        ones = jnp.ones((_LANES,), jnp.float32)
        GUN = 8

        def body(g, carry):
            base = g * (GUN * _LANES)
            idxs = []
            for u in range(GUN):
                o = base + u * _LANES
                idxs.append(lanebase + seg_v[pl.ds(o, _LANES)])
            for q, ch in enumerate((ch0, ch1, ch2, None)):
                qb = q * qstride
                for u in range(GUN):
                    o = base + u * _LANES
                    val = ones if ch is None else ch[pl.ds(o, _LANES)]
                    plsc.addupdate_scatter(acc, [idxs[u] + qb], val)
            return carry

        half_iters = groups // (2 * GUN)
        for cp in cps_a:
            cp.wait()
        lax.fori_loop(0, half_iters, body, 0)
        for cp in cps_b:
            cp.wait()
        lax.fori_loop(half_iters, groups // GUN, body, 0)

        for q in range(_NQ):
            qb = q * qstride
            for t in range(_SEGP // _LANES):
                if t < live // _LANES:
                    vs = [acc[pl.ds(qb + l * rowstr + t * _LANES, _LANES)]
                          for l in range(_LANES)]
                    while len(vs) > 1:
                        vs = [vs[i] + vs[i + 1] for i in range(0, len(vs), 2)]
                    outv[pl.ds(q * _SEGP + t * _LANES, _LANES)] = vs[0]
                else:
                    outv[pl.ds(q * _SEGP + t * _LANES, _LANES)] = zeros

        pltpu.sync_copy(outv, out_hbm.at[pl.ds(wid * outsz, outsz)])

    return sc_kernel(img_flat, seg_flat)


def _tc_fuse_weights(W_proj, b_proj2, W_gcn, b_gcn2):
    def tc_kernel(wp_ref, bp_ref, wg_ref, bg_ref, o_ref):
        wg = wg_ref[...]
        o_ref[0:3, :] = jnp.dot(wp_ref[...], wg,
                                preferred_element_type=jnp.float32)
        o_ref[3:4, :] = jnp.dot(bp_ref[...], wg,
                                preferred_element_type=jnp.float32) + bg_ref[...]

    return pl.pallas_call(
        tc_kernel,
        out_shape=jax.ShapeDtypeStruct((4, _EMB), jnp.float32),
    )(W_proj, b_proj2, W_gcn, b_gcn2)


def _tc_finish(part, wf, B):
    def tc_kernel(part_ref, wf_ref, out_ref):
        t = jnp.sum(part_ref[...], axis=1)
        cnt = jnp.clip(t[:, 3 * _SEGP:4 * _SEGP], 1.0, None)
        u = jnp.broadcast_to(wf_ref[3:4, :], (B, _EMB))
        for c in range(3):
            m = t[:, c * _SEGP:(c + 1) * _SEGP] / cnt
            a_c = jnp.sum(m, axis=1, keepdims=True) / _NSEG
            u = u + a_c * wf_ref[c:c + 1, :]
        out_ref[...] = jnp.broadcast_to(u[:, None, :], out_ref.shape)

    return pl.pallas_call(
        tc_kernel,
        out_shape=jax.ShapeDtypeStruct((B, _NSEG, _EMB), jnp.float32),
    )(part, wf)


def kernel(img, segments, W_proj, b_proj, W_gcn, b_gcn):
    B, C, H, W = img.shape
    HW = H * W
    img_flat = img.reshape(B * C * HW)
    seg_flat = segments.reshape(B * HW).astype(jnp.int32)
    wf = _tc_fuse_weights(W_proj, b_proj.reshape(1, _EMB),
                          W_gcn, b_gcn.reshape(1, _EMB))
    part = _sc_segment_partials(img_flat, seg_flat, B, C, HW)
    part = part.reshape(B, _NW // B, _NQ * _SEGP)
    out = _tc_finish(part, wf, B)
    return out


# speedup vs baseline: 1.2210x; 1.0781x over previous
import functools

import jax
import jax.numpy as jnp
from jax import lax
from jax.experimental import pallas as pl
from jax.experimental.pallas import tpu as pltpu
from jax.experimental.pallas import tpu_sc as plsc

_NSEG = 196
_SEGP = 256
_NQ = 4
_LANES = 16
_NW = 32
_EMB = 768


def _sc_segment_partials(img_flat, seg_flat, B, C, HW):
    ppw = (B * HW) // _NW
    wpb = _NW // B
    groups = ppw // _LANES
    rowstr = 264
    live = 208
    qstride = _LANES * rowstr
    accsz = _NQ * qstride
    outsz = _NQ * _SEGP

    mesh = plsc.VectorSubcoreMesh(
        core_axis_name="c", subcore_axis_name="s", num_cores=2, num_subcores=16)

    @functools.partial(
        pl.kernel,
        mesh=mesh,
        compiler_params=pltpu.CompilerParams(needs_layout_passes=False),
        out_type=jax.ShapeDtypeStruct((_NW * outsz,), jnp.float32),
        scratch_types=[
            pltpu.VMEM((ppw,), jnp.int32),
            pltpu.VMEM((ppw,), jnp.float32),
            pltpu.VMEM((ppw,), jnp.float32),
            pltpu.VMEM((ppw,), jnp.float32),
            pltpu.VMEM((accsz,), jnp.float32),
            pltpu.VMEM((outsz,), jnp.float32),
            pltpu.SemaphoreType.DMA,
            pltpu.SemaphoreType.DMA,
        ],
    )
    def sc_kernel(img_hbm, seg_hbm, out_hbm, seg_v, ch0, ch1, ch2, acc, outv,
                  sem_a, sem_b):
        cid = lax.axis_index("c")
        sid = lax.axis_index("s")
        wid = sid * 2 + cid
        b = wid // wpb
        j = wid % wpb
        pix0 = j * ppw
        hp = ppw // 2
        cps_a = [
            pltpu.async_copy(seg_hbm.at[pl.ds(b * HW + pix0, hp)],
                             seg_v.at[pl.ds(0, hp)], sem_a),
            pltpu.async_copy(img_hbm.at[pl.ds((b * C + 0) * HW + pix0, hp)],
                             ch0.at[pl.ds(0, hp)], sem_a),
            pltpu.async_copy(img_hbm.at[pl.ds((b * C + 1) * HW + pix0, hp)],
                             ch1.at[pl.ds(0, hp)], sem_a),
            pltpu.async_copy(img_hbm.at[pl.ds((b * C + 2) * HW + pix0, hp)],
                             ch2.at[pl.ds(0, hp)], sem_a),
        ]
        cps_b = [
            pltpu.async_copy(seg_hbm.at[pl.ds(b * HW + pix0 + hp, hp)],
                             seg_v.at[pl.ds(hp, hp)], sem_b),
            pltpu.async_copy(img_hbm.at[pl.ds((b * C + 0) * HW + pix0 + hp, hp)],
                             ch0.at[pl.ds(hp, hp)], sem_b),
            pltpu.async_copy(img_hbm.at[pl.ds((b * C + 1) * HW + pix0 + hp, hp)],
                             ch1.at[pl.ds(hp, hp)], sem_b),
            pltpu.async_copy(img_hbm.at[pl.ds((b * C + 2) * HW + pix0 + hp, hp)],
                             ch2.at[pl.ds(hp, hp)], sem_b),
        ]

        zeros = jnp.zeros((_LANES,), jnp.float32)

        for q in range(_NQ):
            for t in range(live // _LANES):
                acc[pl.ds(q * _SEGP + t * _LANES, _LANES)] = zeros

        ones = jnp.ones((_LANES,), jnp.float32)
        GUN = 8

        def body(g, carry):
            base = g * (GUN * _LANES)
            idxs = []
            for u in range(GUN):
                o = base + u * _LANES
                idxs.append(seg_v[pl.ds(o, _LANES)])
            for q, ch in enumerate((ch0, ch1, ch2, None)):
                qb = q * _SEGP
                for u in range(GUN):
                    o = base + u * _LANES
                    val = ones if ch is None else ch[pl.ds(o, _LANES)]
                    plsc.addupdate_scatter(acc, [idxs[u] + qb], val)
            return carry

        half_iters = groups // (2 * GUN)
        for cp in cps_a:
            cp.wait()
        lax.fori_loop(0, half_iters, body, 0)
        for cp in cps_b:
            cp.wait()
        lax.fori_loop(half_iters, groups // GUN, body, 0)

        for q in range(_NQ):
            for t in range(_SEGP // _LANES):
                if t < live // _LANES:
                    outv[pl.ds(q * _SEGP + t * _LANES, _LANES)] = (
                        acc[pl.ds(q * _SEGP + t * _LANES, _LANES)])
                else:
                    outv[pl.ds(q * _SEGP + t * _LANES, _LANES)] = zeros

        pltpu.sync_copy(outv, out_hbm.at[pl.ds(wid * outsz, outsz)])

    return sc_kernel(img_flat, seg_flat)


def _tc_fuse_weights(W_proj, b_proj2, W_gcn, b_gcn2):
    def tc_kernel(wp_ref, bp_ref, wg_ref, bg_ref, o_ref):
        wg = wg_ref[...]
        o_ref[0:3, :] = jnp.dot(wp_ref[...], wg,
                                preferred_element_type=jnp.float32)
        o_ref[3:4, :] = jnp.dot(bp_ref[...], wg,
                                preferred_element_type=jnp.float32) + bg_ref[...]

    return pl.pallas_call(
        tc_kernel,
        out_shape=jax.ShapeDtypeStruct((4, _EMB), jnp.float32),
    )(W_proj, b_proj2, W_gcn, b_gcn2)


def _tc_finish(part, wf, B):
    def tc_kernel(part_ref, wf_ref, out_ref):
        t = jnp.sum(part_ref[...], axis=1)
        cnt = jnp.clip(t[:, 3 * _SEGP:4 * _SEGP], 1.0, None)
        u = jnp.broadcast_to(wf_ref[3:4, :], (B, _EMB))
        for c in range(3):
            m = t[:, c * _SEGP:(c + 1) * _SEGP] / cnt
            a_c = jnp.sum(m, axis=1, keepdims=True) / _NSEG
            u = u + a_c * wf_ref[c:c + 1, :]
        out_ref[...] = jnp.broadcast_to(u[:, None, :], out_ref.shape)

    return pl.pallas_call(
        tc_kernel,
        out_shape=jax.ShapeDtypeStruct((B, _NSEG, _EMB), jnp.float32),
    )(part, wf)


def kernel(img, segments, W_proj, b_proj, W_gcn, b_gcn):
    B, C, H, W = img.shape
    HW = H * W
    img_flat = img.reshape(B * C * HW)
    seg_flat = segments.reshape(B * HW).astype(jnp.int32)
    wf = _tc_fuse_weights(W_proj, b_proj.reshape(1, _EMB),
                          W_gcn, b_gcn.reshape(1, _EMB))
    part = _sc_segment_partials(img_flat, seg_flat, B, C, HW)
    part = part.reshape(B, _NW // B, _NQ * _SEGP)
    out = _tc_finish(part, wf, B)
    return out


# speedup vs baseline: 1.2224x; 1.0012x over previous
import functools

import jax
import jax.numpy as jnp
from jax import lax
from jax.experimental import pallas as pl
from jax.experimental.pallas import tpu as pltpu
from jax.experimental.pallas import tpu_sc as plsc

_NSEG = 196
_SEGP = 256
_NQ = 4
_LANES = 16
_NW = 32
_EMB = 768


def _sc_segment_partials(img_flat, seg_flat, B, C, HW):
    ppw = (B * HW) // _NW
    wpb = _NW // B
    groups = ppw // _LANES
    outsz = _NQ * _SEGP
    accsz = outsz

    mesh = plsc.VectorSubcoreMesh(
        core_axis_name="c", subcore_axis_name="s", num_cores=2, num_subcores=16)

    @functools.partial(
        pl.kernel,
        mesh=mesh,
        compiler_params=pltpu.CompilerParams(needs_layout_passes=False),
        out_type=jax.ShapeDtypeStruct((_NW * outsz,), jnp.float32),
        scratch_types=[
            pltpu.VMEM((ppw,), jnp.int32),
            pltpu.VMEM((ppw,), jnp.float32),
            pltpu.VMEM((ppw,), jnp.float32),
            pltpu.VMEM((ppw,), jnp.float32),
            pltpu.VMEM((accsz,), jnp.float32),
            pltpu.SemaphoreType.DMA,
            pltpu.SemaphoreType.DMA,
        ],
    )
    def sc_kernel(img_hbm, seg_hbm, out_hbm, seg_v, ch0, ch1, ch2, acc,
                  sem_a, sem_b):
        cid = lax.axis_index("c")
        sid = lax.axis_index("s")
        wid = sid * 2 + cid
        b = wid // wpb
        j = wid % wpb
        pix0 = j * ppw
        hp = ppw // 2
        cps_a = [
            pltpu.async_copy(seg_hbm.at[pl.ds(b * HW + pix0, hp)],
                             seg_v.at[pl.ds(0, hp)], sem_a),
            pltpu.async_copy(img_hbm.at[pl.ds((b * C + 0) * HW + pix0, hp)],
                             ch0.at[pl.ds(0, hp)], sem_a),
            pltpu.async_copy(img_hbm.at[pl.ds((b * C + 1) * HW + pix0, hp)],
                             ch1.at[pl.ds(0, hp)], sem_a),
            pltpu.async_copy(img_hbm.at[pl.ds((b * C + 2) * HW + pix0, hp)],
                             ch2.at[pl.ds(0, hp)], sem_a),
        ]
        cps_b = [
            pltpu.async_copy(seg_hbm.at[pl.ds(b * HW + pix0 + hp, hp)],
                             seg_v.at[pl.ds(hp, hp)], sem_b),
            pltpu.async_copy(img_hbm.at[pl.ds((b * C + 0) * HW + pix0 + hp, hp)],
                             ch0.at[pl.ds(hp, hp)], sem_b),
            pltpu.async_copy(img_hbm.at[pl.ds((b * C + 1) * HW + pix0 + hp, hp)],
                             ch1.at[pl.ds(hp, hp)], sem_b),
            pltpu.async_copy(img_hbm.at[pl.ds((b * C + 2) * HW + pix0 + hp, hp)],
                             ch2.at[pl.ds(hp, hp)], sem_b),
        ]

        zeros = jnp.zeros((_LANES,), jnp.float32)

        for q in range(_NQ):
            for t in range(_SEGP // _LANES):
                acc[pl.ds(q * _SEGP + t * _LANES, _LANES)] = zeros

        ones = jnp.ones((_LANES,), jnp.float32)
        GUN = 8

        def body(g, carry):
            base = g * (GUN * _LANES)
            idxs = []
            for u in range(GUN):
                o = base + u * _LANES
                idxs.append(seg_v[pl.ds(o, _LANES)])
            for q, ch in enumerate((ch0, ch1, ch2, None)):
                qb = q * _SEGP
                for u in range(GUN):
                    o = base + u * _LANES
                    val = ones if ch is None else ch[pl.ds(o, _LANES)]
                    plsc.addupdate_scatter(acc, [idxs[u] + qb], val)
            return carry

        half_iters = groups // (2 * GUN)
        for cp in cps_a:
            cp.wait()
        lax.fori_loop(0, half_iters, body, 0)
        for cp in cps_b:
            cp.wait()
        lax.fori_loop(half_iters, groups // GUN, body, 0)

        pltpu.sync_copy(acc.at[pl.ds(0, outsz)],
                        out_hbm.at[pl.ds(wid * outsz, outsz)])

    return sc_kernel(img_flat, seg_flat)


def _tc_fuse_weights(W_proj, b_proj2, W_gcn, b_gcn2):
    def tc_kernel(wp_ref, bp_ref, wg_ref, bg_ref, o_ref):
        wg = wg_ref[...]
        o_ref[0:3, :] = jnp.dot(wp_ref[...], wg,
                                preferred_element_type=jnp.float32)
        o_ref[3:4, :] = jnp.dot(bp_ref[...], wg,
                                preferred_element_type=jnp.float32) + bg_ref[...]

    return pl.pallas_call(
        tc_kernel,
        out_shape=jax.ShapeDtypeStruct((4, _EMB), jnp.float32),
    )(W_proj, b_proj2, W_gcn, b_gcn2)


def _tc_finish(part, wf, B):
    def tc_kernel(part_ref, wf_ref, out_ref):
        t = jnp.sum(part_ref[...], axis=1)
        cnt = jnp.clip(t[:, 3 * _SEGP:4 * _SEGP], 1.0, None)
        u = jnp.broadcast_to(wf_ref[3:4, :], (B, _EMB))
        for c in range(3):
            m = t[:, c * _SEGP:(c + 1) * _SEGP] / cnt
            a_c = jnp.sum(m, axis=1, keepdims=True) / _NSEG
            u = u + a_c * wf_ref[c:c + 1, :]
        out_ref[...] = jnp.broadcast_to(u[:, None, :], out_ref.shape)

    return pl.pallas_call(
        tc_kernel,
        out_shape=jax.ShapeDtypeStruct((B, _NSEG, _EMB), jnp.float32),
    )(part, wf)


def kernel(img, segments, W_proj, b_proj, W_gcn, b_gcn):
    B, C, H, W = img.shape
    HW = H * W
    img_flat = img.reshape(B * C * HW)
    seg_flat = segments.reshape(B * HW).astype(jnp.int32)
    wf = _tc_fuse_weights(W_proj, b_proj.reshape(1, _EMB),
                          W_gcn, b_gcn.reshape(1, _EMB))
    part = _sc_segment_partials(img_flat, seg_flat, B, C, HW)
    part = part.reshape(B, _NW // B, _NQ * _SEGP)
    out = _tc_finish(part, wf, B)
    return out
